# baseline (speedup 1.0000x reference)
import functools

import jax
import jax.numpy as jnp
from jax import lax
from jax.experimental import pallas as pl
from jax.experimental.pallas import tpu as pltpu
from jax.experimental.pallas import tpu_sc as plsc

_NC = 2
_NS = 16
_NW = _NC * _NS
_CB = 8


def _sc_detile(tableT):
    Dd, V = tableT.shape
    NT = V // 128
    TAIL = V - NT * 128
    KMAX = -(-NT // _NW)
    NPAIR = -(-KMAX // 2)

    mesh = plsc.VectorSubcoreMesh(core_axis_name="c", subcore_axis_name="s")

    @functools.partial(
        pl.kernel,
        mesh=mesh,
        compiler_params=pltpu.CompilerParams(
            use_tc_tiling_on_sc=False, needs_layout_passes=False
        ),
        out_type=jax.ShapeDtypeStruct((V * Dd,), jnp.float32),
        scratch_types=[
            pltpu.VMEM((2, 16, 128), jnp.float32),
            pltpu.VMEM((2, 128 * 16), jnp.float32),
            pltpu.SemaphoreType.DMA,
            pltpu.SemaphoreType.DMA,
            pltpu.SemaphoreType.DMA,
            pltpu.SemaphoreType.DMA,
        ],
    )
    def detile_kernel(tt_hbm, out_hbm, ibuf, obuf, isem0, isem1, osem0, osem1):
        wid = lax.axis_index("s") * _NC + lax.axis_index("c")
        isems = (isem0, isem1)
        osems = (osem0, osem1)
        i0 = jnp.arange(16, dtype=jnp.int32)

        def fire_in(t, b):
            pltpu.async_copy(tt_hbm.at[:, pl.ds(t * 128, 128)], ibuf.at[b], isems[b])

        def wait_in(t, b):
            pltpu.make_async_copy(tt_hbm.at[:, pl.ds(t * 128, 128)], ibuf.at[b], isems[b]).wait()

        def transform(b):
            for r in range(128):
                v = plsc.load_gather(ibuf.at[b], [i0, jnp.full((16,), r, jnp.int32)])
                obuf[b, pl.ds(r * 16, 16)] = v

        def fire_out(t, b):
            pltpu.async_copy(obuf.at[b], out_hbm.at[pl.ds(t * 2048, 2048)], osems[b])

        def wait_out(t, b):
            pltpu.make_async_copy(obuf.at[b], out_hbm.at[pl.ds(t * 2048, 2048)], osems[b]).wait()

        def t_of(k):
            return wid + _NW * k

        @pl.when(t_of(0) < NT)
        def _():
            fire_in(t_of(0), 0)

        @pl.when(t_of(1) < NT)
        def _():
            fire_in(t_of(1), 1)

        def step(p, k, b):
            t = t_of(k)

            @pl.when(t < NT)
            def _():
                wait_in(t, b)

                @pl.when(p > 0)
                def _():
                    wait_out(t_of(k - 2), b)

                transform(b)
                fire_out(t, b)

                @pl.when(t_of(k + 2) < NT)
                def _():
                    fire_in(t_of(k + 2), b)

        def pair_body(p, carry):
            step(p, 2 * p, 0)
            step(p, 2 * p + 1, 1)
            return carry

        lax.fori_loop(0, NPAIR, pair_body, 0)

        for b in (0, 1):
            kb = KMAX - 1 - ((KMAX - 1 - b) % 2)

            @pl.when(t_of(kb) < NT)
            def _(kb=kb, b=b):
                wait_out(t_of(kb), b)

            @pl.when(t_of(kb) >= NT)
            def _(kb=kb, b=b):
                wait_out(t_of(kb - 2), b)

        if TAIL:
            @pl.when(wid == NT % _NW)
            def _():
                l0 = NT * 128
                pltpu.sync_copy(tt_hbm.at[:, pl.ds(l0, TAIL)], ibuf.at[0, :, pl.ds(0, TAIL)])
                for r in range(TAIL):
                    v = plsc.load_gather(ibuf.at[0], [i0, jnp.full((16,), r, jnp.int32)])
                    obuf[0, pl.ds(r * 16, 16)] = v
                pltpu.sync_copy(obuf.at[0, pl.ds(0, TAIL * 16)], out_hbm.at[pl.ds(l0 * 16, TAIL * 16)])

    return detile_kernel(tableT)


def _sc_pool(x, table):
    B, H = x.shape
    HA = 104
    HB = H - HA
    V, D = table.shape
    per_w = B // _NW
    n_chunks = per_w // _CB
    n_pairs = n_chunks // 2
    inv_h = jnp.float32(1.0 / H)

    mesh = plsc.VectorSubcoreMesh(core_axis_name="c", subcore_axis_name="s")

    @functools.partial(
        pl.kernel,
        mesh=mesh,
        compiler_params=pltpu.CompilerParams(use_tc_tiling_on_sc=False),
        out_type=jax.ShapeDtypeStruct((B, D), jnp.float32),
        scratch_types=[
            pltpu.VMEM((2, _CB, HA), jnp.int32),
            pltpu.VMEM((2, _CB, HB), jnp.int32),
            pltpu.VMEM((2, _CB, H, D), jnp.float32),
            pltpu.VMEM((_CB, D), jnp.float32),
            pltpu.SemaphoreType.DMA,
            pltpu.SemaphoreType.DMA,
        ],
    )
    def pool_kernel(x_hbm, table_hbm, out_hbm, idx_a, idx_b, rows_v, pooled_v, sem0, sem1):
        wid = lax.axis_index("s") * _NC + lax.axis_index("c")
        base = wid * per_w
        sems = (sem0, sem1)

        def load_and_fire(t, buf):
            r0 = base + t * _CB
            pltpu.sync_copy(x_hbm.at[pl.ds(r0, _CB), pl.ds(0, HA)], idx_a.at[buf])
            pltpu.sync_copy(x_hbm.at[pl.ds(r0, _CB), pl.ds(HA, HB)], idx_b.at[buf])
            for e in range(_CB):
                pltpu.async_copy(
                    table_hbm.at[idx_a.at[buf, e]],
                    rows_v.at[buf, e, pl.ds(0, HA)],
                    sems[buf],
                )
                pltpu.async_copy(
                    table_hbm.at[idx_b.at[buf, e]],
                    rows_v.at[buf, e, pl.ds(HA, HB)],
                    sems[buf],
                )

        def wait_gathers(buf):
            for e in range(_CB):
                pltpu.make_async_copy(
                    table_hbm.at[idx_a.at[buf, e]],
                    rows_v.at[buf, e, pl.ds(0, HA)],
                    sems[buf],
                ).wait()
                pltpu.make_async_copy(
                    table_hbm.at[idx_b.at[buf, e]],
                    rows_v.at[buf, e, pl.ds(HA, HB)],
                    sems[buf],
                ).wait()

        def reduce_chunk(t, buf):
            for e in range(_CB):
                zero = jnp.zeros((D,), jnp.float32)

                def body(j, accs, _e=e, _buf=buf):
                    a0, a1, a2, a3 = accs
                    j4 = 4 * j
                    a0 = a0 + rows_v[_buf, _e, j4]
                    a1 = a1 + rows_v[_buf, _e, j4 + 1]
                    a2 = a2 + rows_v[_buf, _e, j4 + 2]
                    a3 = a3 + rows_v[_buf, _e, j4 + 3]
                    return (a0, a1, a2, a3)

                a0, a1, a2, a3 = lax.fori_loop(0, H // 4, body, (zero,) * 4)
                pooled_v[e] = ((a0 + a1) + (a2 + a3)) * inv_h
            pltpu.sync_copy(pooled_v, out_hbm.at[pl.ds(base + t * _CB, _CB)])

        load_and_fire(0, 0)

        def pair_body(p, carry):
            t0 = 2 * p
            load_and_fire(t0 + 1, 1)
            wait_gathers(0)
            reduce_chunk(t0, 0)

            @pl.when(p < n_pairs - 1)
            def _():
                load_and_fire(t0 + 2, 0)

            wait_gathers(1)
            reduce_chunk(t0 + 1, 1)
            return carry

        lax.fori_loop(0, n_pairs, pair_body, 0)

    return pool_kernel(x, table)


def _tc_head(pooled, W, b2):
    B, D = pooled.shape
    C = W.shape[1]
    BT = 2048

    def head_body(p_ref, w_ref, b_ref, o_ref):
        logits = (
            jnp.dot(p_ref[...], w_ref[...], preferred_element_type=jnp.float32)
            + b_ref[...]
        )
        m = jnp.max(logits, axis=-1, keepdims=True)
        e = jnp.exp(logits - m)
        o_ref[...] = e / jnp.sum(e, axis=-1, keepdims=True)

    return pl.pallas_call(
        head_body,
        grid=(B // BT,),
        in_specs=[
            pl.BlockSpec((BT, D), lambda i: (i, 0)),
            pl.BlockSpec((D, C), lambda i: (0, 0)),
            pl.BlockSpec((1, C), lambda i: (0, 0)),
        ],
        out_specs=pl.BlockSpec((BT, C), lambda i: (i, 0)),
        out_shape=jax.ShapeDtypeStruct((B, C), jnp.float32),
    )(pooled, W, b2)


def kernel(x, table, W, b):
    V, D = table.shape
    tlin = _sc_detile(table.T)
    pooled = _sc_pool(x, tlin.reshape(V, D))
    return _tc_head(pooled, W, b.reshape(1, -1))


# speedup vs baseline: 2.9849x; 2.9849x over previous
import functools

import jax
import jax.numpy as jnp
from jax import lax
from jax.experimental import pallas as pl
from jax.experimental.pallas import tpu as pltpu
from jax.experimental.pallas import tpu_sc as plsc

_NC = 2
_NS = 16
_NW = _NC * _NS
_CB = 8


def _sc_detile(tableT, tail2):
    Dd, V = tableT.shape
    NT = V // 128
    TAIL = V - NT * 128
    KMAX = -(-NT // _NW)
    NPAIR = -(-KMAX // 2)

    mesh = plsc.VectorSubcoreMesh(core_axis_name="c", subcore_axis_name="s")

    @functools.partial(
        pl.kernel,
        mesh=mesh,
        compiler_params=pltpu.CompilerParams(
            use_tc_tiling_on_sc=True, needs_layout_passes=False
        ),
        out_type=jax.ShapeDtypeStruct((V * Dd // 128, 128), jnp.float32),
        scratch_types=[
            pltpu.VMEM((4, 8, 128), jnp.float32),
            pltpu.VMEM((16, 128), jnp.float32),
            pltpu.VMEM((16, 128), jnp.float32),
            pltpu.SemaphoreType.DMA,
            pltpu.SemaphoreType.DMA,
            pltpu.SemaphoreType.DMA,
            pltpu.SemaphoreType.DMA,
        ],
    )
    def detile_kernel(tt_hbm, tail_hbm, out_hbm, ibuf, obuf0, obuf1, isem0, isem1, osem0, osem1):
        wid = lax.axis_index("s") * _NC + lax.axis_index("c")
        isems = (isem0, isem1)
        osems = (osem0, osem1)
        obufs = (obuf0, obuf1)
        lane = jnp.arange(16, dtype=jnp.int32)
        i0 = lane // 8
        i1 = lane % 8

        def fire_in(t, b):
            l0 = t * 128
            pltpu.async_copy(tt_hbm.at[pl.ds(0, 8), pl.ds(l0, 128)], ibuf.at[2 * b], isems[b])
            pltpu.async_copy(tt_hbm.at[pl.ds(8, 8), pl.ds(l0, 128)], ibuf.at[2 * b + 1], isems[b])

        def wait_in(t, b):
            l0 = t * 128
            pltpu.make_async_copy(tt_hbm.at[pl.ds(0, 8), pl.ds(l0, 128)], ibuf.at[2 * b], isems[b]).wait()
            pltpu.make_async_copy(tt_hbm.at[pl.ds(8, 8), pl.ds(l0, 128)], ibuf.at[2 * b + 1], isems[b]).wait()

        def transform(b):
            for r in range(128):
                v = plsc.load_gather(
                    ibuf.at[pl.ds(2 * b, 2)], [i0, i1, jnp.full((16,), r, jnp.int32)]
                )
                obufs[b][r // 8, pl.ds((r % 8) * 16, 16)] = v

        def fire_out(t, b):
            pltpu.async_copy(obufs[b], out_hbm.at[pl.ds(t * 16, 16)], osems[b])

        def wait_out(t, b):
            pltpu.make_async_copy(obufs[b], out_hbm.at[pl.ds(t * 16, 16)], osems[b]).wait()

        def t_of(k):
            return wid + _NW * k

        @pl.when(t_of(0) < NT)
        def _():
            fire_in(t_of(0), 0)

        @pl.when(t_of(1) < NT)
        def _():
            fire_in(t_of(1), 1)

        def step(p, k, b):
            t = t_of(k)

            @pl.when(t < NT)
            def _():
                wait_in(t, b)

                @pl.when(p > 0)
                def _():
                    wait_out(t_of(k - 2), b)

                transform(b)
                fire_out(t, b)

                @pl.when(t_of(k + 2) < NT)
                def _():
                    fire_in(t_of(k + 2), b)

        def pair_body(p, carry):
            step(p, 2 * p, 0)
            step(p, 2 * p + 1, 1)
            return carry

        lax.fori_loop(0, NPAIR, pair_body, 0)

        for b in (0, 1):
            kb = KMAX - 1 - ((KMAX - 1 - b) % 2)

            @pl.when(t_of(kb) < NT)
            def _(kb=kb, b=b):
                wait_out(t_of(kb), b)

            @pl.when(t_of(kb) >= NT)
            def _(kb=kb, b=b):
                wait_out(t_of(kb - 2), b)

        if TAIL:
            TR = TAIL * Dd // 128

            @pl.when(wid == NT % _NW)
            def _():
                pltpu.sync_copy(tail_hbm, ibuf.at[0])
                pltpu.sync_copy(ibuf.at[0], out_hbm.at[pl.ds(NT * 16, TR)])

    return detile_kernel(tableT, tail2)


def _sc_pool(x, table):
    B, H = x.shape
    HA = 104
    HB = H - HA
    V, D = table.shape
    per_w = B // _NW
    n_chunks = per_w // _CB
    n_pairs = n_chunks // 2
    inv_h = jnp.float32(1.0 / H)

    mesh = plsc.VectorSubcoreMesh(core_axis_name="c", subcore_axis_name="s")

    @functools.partial(
        pl.kernel,
        mesh=mesh,
        compiler_params=pltpu.CompilerParams(use_tc_tiling_on_sc=False),
        out_type=jax.ShapeDtypeStruct((B, D), jnp.float32),
        scratch_types=[
            pltpu.VMEM((2, _CB, HA), jnp.int32),
            pltpu.VMEM((2, _CB, HB), jnp.int32),
            pltpu.VMEM((2, _CB, H, D), jnp.float32),
            pltpu.VMEM((_CB, D), jnp.float32),
            pltpu.SemaphoreType.DMA,
            pltpu.SemaphoreType.DMA,
        ],
    )
    def pool_kernel(x_hbm, table_hbm, out_hbm, idx_a, idx_b, rows_v, pooled_v, sem0, sem1):
        wid = lax.axis_index("s") * _NC + lax.axis_index("c")
        base = wid * per_w
        sems = (sem0, sem1)

        def load_and_fire(t, buf):
            r0 = base + t * _CB
            pltpu.sync_copy(x_hbm.at[pl.ds(r0, _CB), pl.ds(0, HA)], idx_a.at[buf])
            pltpu.sync_copy(x_hbm.at[pl.ds(r0, _CB), pl.ds(HA, HB)], idx_b.at[buf])
            for e in range(_CB):
                pltpu.async_copy(
                    table_hbm.at[idx_a.at[buf, e]],
                    rows_v.at[buf, e, pl.ds(0, HA)],
                    sems[buf],
                )
                pltpu.async_copy(
                    table_hbm.at[idx_b.at[buf, e]],
                    rows_v.at[buf, e, pl.ds(HA, HB)],
                    sems[buf],
                )

        def wait_gathers(buf):
            for e in range(_CB):
                pltpu.make_async_copy(
                    table_hbm.at[idx_a.at[buf, e]],
                    rows_v.at[buf, e, pl.ds(0, HA)],
                    sems[buf],
                ).wait()
                pltpu.make_async_copy(
                    table_hbm.at[idx_b.at[buf, e]],
                    rows_v.at[buf, e, pl.ds(HA, HB)],
                    sems[buf],
                ).wait()

        def reduce_chunk(t, buf):
            for e in range(_CB):
                zero = jnp.zeros((D,), jnp.float32)

                def body(j, accs, _e=e, _buf=buf):
                    a0, a1, a2, a3 = accs
                    j4 = 4 * j
                    a0 = a0 + rows_v[_buf, _e, j4]
                    a1 = a1 + rows_v[_buf, _e, j4 + 1]
                    a2 = a2 + rows_v[_buf, _e, j4 + 2]
                    a3 = a3 + rows_v[_buf, _e, j4 + 3]
                    return (a0, a1, a2, a3)

                a0, a1, a2, a3 = lax.fori_loop(0, H // 4, body, (zero,) * 4)
                pooled_v[e] = ((a0 + a1) + (a2 + a3)) * inv_h
            pltpu.sync_copy(pooled_v, out_hbm.at[pl.ds(base + t * _CB, _CB)])

        load_and_fire(0, 0)

        def pair_body(p, carry):
            t0 = 2 * p
            load_and_fire(t0 + 1, 1)
            wait_gathers(0)
            reduce_chunk(t0, 0)

            @pl.when(p < n_pairs - 1)
            def _():
                load_and_fire(t0 + 2, 0)

            wait_gathers(1)
            reduce_chunk(t0 + 1, 1)
            return carry

        lax.fori_loop(0, n_pairs, pair_body, 0)

    return pool_kernel(x, table)


def _tc_head(pooled, W, b2):
    B, D = pooled.shape
    C = W.shape[1]
    BT = 2048

    def head_body(p_ref, w_ref, b_ref, o_ref):
        logits = (
            jnp.dot(p_ref[...], w_ref[...], preferred_element_type=jnp.float32)
            + b_ref[...]
        )
        m = jnp.max(logits, axis=-1, keepdims=True)
        e = jnp.exp(logits - m)
        o_ref[...] = e / jnp.sum(e, axis=-1, keepdims=True)

    return pl.pallas_call(
        head_body,
        grid=(B // BT,),
        in_specs=[
            pl.BlockSpec((BT, D), lambda i: (i, 0)),
            pl.BlockSpec((D, C), lambda i: (0, 0)),
            pl.BlockSpec((1, C), lambda i: (0, 0)),
        ],
        out_specs=pl.BlockSpec((BT, C), lambda i: (i, 0)),
        out_shape=jax.ShapeDtypeStruct((B, C), jnp.float32),
    )(pooled, W, b2)


def kernel(x, table, W, b):
    V, D = table.shape
    nt = (V // 128) * 128
    tail2 = jax.lax.slice(table, (nt, 0), (V, D)).reshape(-1, 128)
    tlin = _sc_detile(table.T, tail2)
    pooled = _sc_pool(x, tlin.reshape(V, D))
    return _tc_head(pooled, W, b.reshape(1, -1))
